# Initial kernel scaffold; baseline (speedup 1.0000x reference)
import jax
import jax.numpy as jnp
from jax.experimental import pallas as pl


def kernel(x, W, b, gate_W, gate_b, expert_biases):
    raise NotImplementedError("write your pallas kernel here")


# speedup vs baseline: 10.1661x; 10.1661x over previous
import functools

import jax
import jax.numpy as jnp
from jax import lax
from jax.experimental import pallas as pl

INTERPRET = False


def _moe_block(x_ref, c_ref, bias_ref, out_ref, idx_ref, *, T, O, E):
    scores = jnp.dot(x_ref[...], c_ref[...], preferred_element_type=jnp.float32)
    scores = scores + bias_ref[...][0][None, :]
    logits = scores[:, 0:E]
    col = lax.broadcasted_iota(jnp.int32, (T, E), 1)
    m0 = jnp.max(logits, axis=1)
    i0 = jnp.min(jnp.where(logits == m0[:, None], col, E), axis=1)
    masked = jnp.where(col == i0[:, None], -1e30, logits)
    m1 = jnp.max(masked, axis=1)
    i1 = jnp.min(jnp.where(masked == m1[:, None], col, E), axis=1)
    p0 = 1.0 / (1.0 + jnp.exp(-m0))
    p1 = 1.0 / (1.0 + jnp.exp(-m1))
    v0 = jnp.sum(jnp.where(col == i0[:, None], scores[:, E:2 * E], 0.0), axis=1)
    v1 = jnp.sum(jnp.where(col == i1[:, None], scores[:, 2 * E:3 * E], 0.0), axis=1)
    outs = (p0 * v0 + p1 * v1) / (p0 + p1)
    out_ref[...] = jnp.broadcast_to(outs[:, None], (T, O))
    idx_ref[...] = jnp.concatenate([i0[:, None], i1[:, None]], axis=1)


def kernel(x, W, b, gate_W, gate_b, expert_biases):
    k = 2
    B, S, D = x.shape
    E, O, _ = W.shape
    tokens = B * S
    xf = x.reshape(tokens, D)
    width = 4 * E
    C = jnp.concatenate(
        [gate_W, W[:, 0, :], W[:, 1, :], jnp.zeros((E, D), jnp.float32)], axis=0
    ).T
    bias = jnp.concatenate(
        [gate_b + expert_biases, b[:, 0], b[:, 1], jnp.zeros((E,), jnp.float32)]
    )
    bias_arr = jnp.tile(bias[None, :], (8, 1))

    T = min(512, tokens)
    grid = (tokens // T,)
    out, idx = pl.pallas_call(
        functools.partial(_moe_block, T=T, O=O, E=E),
        grid=grid,
        in_specs=[
            pl.BlockSpec((T, D), lambda i: (i, 0)),
            pl.BlockSpec((D, width), lambda i: (0, 0)),
            pl.BlockSpec((8, width), lambda i: (0, 0)),
        ],
        out_specs=[
            pl.BlockSpec((T, O), lambda i: (i, 0)),
            pl.BlockSpec((T, k), lambda i: (i, 0)),
        ],
        out_shape=[
            jax.ShapeDtypeStruct((tokens, O), jnp.float32),
            jax.ShapeDtypeStruct((tokens, k), jnp.int32),
        ],
        interpret=INTERPRET,
    )(xf, C, bias_arr)
    return out.reshape(B, S, O), idx.reshape(B, S, k)
